# Initial kernel scaffold; baseline (speedup 1.0000x reference)
import jax
import jax.numpy as jnp
from jax.experimental import pallas as pl


def kernel(embed, sigma, anchor_class, pos_class, neg_class, anchor_ingred, pos_ingred, neg_ingred):
    raise NotImplementedError("write your pallas kernel here")


# speedup vs baseline: 2.4151x; 2.4151x over previous
import functools

import jax
import jax.numpy as jnp
from jax import lax
from jax.experimental import pallas as pl
from jax.experimental.pallas import tpu as pltpu
from jax.experimental.pallas import tpu_sc as plsc


@functools.lru_cache(maxsize=None)
def _make_sc_gather(total, D):
    info = plsc.get_sparse_core_info()
    NC, NS = info.num_cores, info.num_subcores
    NW = NC * NS
    CH = 128
    assert total % (NW * CH) == 0
    n_g = total // (NW * CH)
    b_per_w = n_g * CH

    mesh = plsc.VectorSubcoreMesh(core_axis_name="c", subcore_axis_name="s")

    @functools.partial(
        pl.kernel,
        mesh=mesh,
        out_type=jax.ShapeDtypeStruct((total, D), jnp.float32),
        scratch_types=[
            pltpu.VMEM((n_g, CH), jnp.int32),
            pltpu.VMEM((CH, D), jnp.float32),
            pltpu.VMEM((CH, D), jnp.float32),
            pltpu.SemaphoreType.DMA,
            pltpu.SemaphoreType.DMA,
        ],
    )
    def gather_k(table_hbm, idx_hbm, out_hbm, idx_v, rows0, rows1, sem0, sem1):
        wid = lax.axis_index("s") * NC + lax.axis_index("c")
        base = wid * b_per_w
        pltpu.sync_copy(idx_hbm.at[wid], idx_v)
        bufs = (rows0, rows1)
        sems = (sem0, sem1)
        cps = [None, None]
        cps[0] = pltpu.async_copy(table_hbm.at[idx_v.at[0]], bufs[0], sems[0])
        for g in range(n_g):
            if g + 1 < n_g:
                cps[(g + 1) % 2] = pltpu.async_copy(
                    table_hbm.at[idx_v.at[g + 1]], bufs[(g + 1) % 2],
                    sems[(g + 1) % 2])
            cps[g % 2].wait()
            pltpu.sync_copy(bufs[g % 2], out_hbm.at[pl.ds(base + g * CH, CH)])

    def run(table, idx):
        idx3 = idx.reshape(NW, n_g, CH)
        return gather_k(table, idx3)

    return run


def _ntxent_body(sig_ref, a_ref, p_ref, n_ref, out_ref, acc_ref):
    l = pl.program_id(0)
    j = pl.program_id(1)
    nj = pl.num_programs(1)

    a = a_ref[0]
    an = a / jnp.maximum(jnp.sqrt(jnp.sum(a * a, axis=1, keepdims=True)), 1e-12)
    nb = n_ref[0]
    nn = nb / jnp.maximum(jnp.sqrt(jnp.sum(nb * nb, axis=1, keepdims=True)), 1e-12)
    sim = lax.dot_general(an, nn, (((1,), (1,)), ((), ())),
                          preferred_element_type=jnp.float32)
    es = jnp.sum(jnp.exp(sim * 2.0), axis=1, keepdims=True)

    @pl.when(j == 0)
    def _():
        acc_ref[...] = es

    @pl.when(j > 0)
    def _():
        acc_ref[...] += es

    @pl.when(j == nj - 1)
    def _():
        p = p_ref[0]
        pn = p / jnp.maximum(jnp.sqrt(jnp.sum(p * p, axis=1, keepdims=True)),
                             1e-12)
        pos = jnp.sum(an * pn, axis=1, keepdims=True) * 2.0
        lse = jnp.log(acc_ref[...] + jnp.exp(pos))
        part = jnp.mean(lse - pos)
        s = jnp.where(l == 0, sig_ref[0, 0], sig_ref[0, 1])
        contrib = part / (2.0 * s * s) + jnp.log(s)

        contrib2d = jnp.reshape(contrib, (1, 1))

        @pl.when(l == 0)
        def _():
            out_ref[...] = contrib2d

        @pl.when(l == 1)
        def _():
            out_ref[...] = out_ref[...] + contrib2d


@functools.lru_cache(maxsize=None)
def _make_ntxent(B, K, D, CB=2048):
    BK = B * K
    assert BK % CB == 0
    nj = BK // CB
    return pl.pallas_call(
        _ntxent_body,
        grid=(2, nj),
        in_specs=[
            pl.BlockSpec((1, 2), lambda l, j: (0, 0)),
            pl.BlockSpec((1, B, D), lambda l, j: (l, 0, 0)),
            pl.BlockSpec((1, B, D), lambda l, j: (l, 0, 0)),
            pl.BlockSpec((1, CB, D), lambda l, j: (l, j, 0)),
        ],
        out_specs=pl.BlockSpec((1, 1), lambda l, j: (0, 0)),
        out_shape=jax.ShapeDtypeStruct((1, 1), jnp.float32),
        scratch_shapes=[pltpu.VMEM((B, 1), jnp.float32)],
        compiler_params=pltpu.CompilerParams(
            dimension_semantics=("arbitrary", "arbitrary")),
    )


def kernel(embed, sigma, anchor_class, pos_class, neg_class,
           anchor_ingred, pos_ingred, neg_ingred):
    D = embed.shape[1]
    B = anchor_class.shape[0]
    K = neg_class.shape[0] // B

    idx = jnp.concatenate([
        anchor_class, anchor_ingred, pos_class, pos_ingred,
        neg_class, neg_ingred,
    ])
    rows = _make_sc_gather(idx.shape[0], D)(embed, idx)

    A = rows[:2 * B].reshape(2, B, D)
    P = rows[2 * B:4 * B].reshape(2, B, D)
    N = rows[4 * B:].reshape(2, B * K, D)

    out = _make_ntxent(B, K, D)(sigma.reshape(1, 2), A, P, N)
    return out.reshape(())


# speedup vs baseline: 2.5701x; 1.0642x over previous
import functools

import jax
import jax.numpy as jnp
from jax import lax
from jax.experimental import pallas as pl
from jax.experimental.pallas import tpu as pltpu
from jax.experimental.pallas import tpu_sc as plsc


@functools.lru_cache(maxsize=None)
def _make_sc_gather(total, D):
    info = plsc.get_sparse_core_info()
    NC, NS = info.num_cores, info.num_subcores
    NW = NC * NS
    CH = 128
    assert total % (NW * CH) == 0
    n_g = total // (NW * CH)
    b_per_w = n_g * CH

    mesh = plsc.VectorSubcoreMesh(core_axis_name="c", subcore_axis_name="s")

    @functools.partial(
        pl.kernel,
        mesh=mesh,
        out_type=jax.ShapeDtypeStruct((total, D), jnp.float32),
        scratch_types=[
            pltpu.VMEM((n_g, CH), jnp.int32),
            pltpu.VMEM((CH, D), jnp.float32),
            pltpu.VMEM((CH, D), jnp.float32),
            pltpu.SemaphoreType.DMA,
            pltpu.SemaphoreType.DMA,
        ],
    )
    def gather_k(table_hbm, idx_hbm, out_hbm, idx_v, rows0, rows1, sem0, sem1):
        wid = lax.axis_index("s") * NC + lax.axis_index("c")
        base = wid * b_per_w
        pltpu.sync_copy(idx_hbm.at[wid], idx_v)
        bufs = (rows0, rows1)
        sems = (sem0, sem1)
        cps = [None, None]
        cps[0] = pltpu.async_copy(table_hbm.at[idx_v.at[0]], bufs[0], sems[0])
        for g in range(n_g):
            if g + 1 < n_g:
                cps[(g + 1) % 2] = pltpu.async_copy(
                    table_hbm.at[idx_v.at[g + 1]], bufs[(g + 1) % 2],
                    sems[(g + 1) % 2])
            cps[g % 2].wait()
            pltpu.sync_copy(bufs[g % 2], out_hbm.at[pl.ds(base + g * CH, CH)])

    def run(table, idx):
        idx3 = idx.reshape(NW, n_g, CH)
        return gather_k(table, idx3)

    return run


def _ntxent_body(sig_ref, a_ref, p_ref, n_ref, out_ref, acc_ref):
    l = pl.program_id(0)
    j = pl.program_id(1)
    nj = pl.num_programs(1)

    LOG2E2 = 2.0 * 1.4426950408889634
    a = a_ref[0]
    an = a * lax.rsqrt(jnp.maximum(jnp.sum(a * a, axis=1, keepdims=True),
                                   1e-24))
    nb = n_ref[0]
    nn = nb * lax.rsqrt(jnp.maximum(jnp.sum(nb * nb, axis=1, keepdims=True),
                                    1e-24))
    sim2 = lax.dot_general(an * LOG2E2, nn, (((1,), (1,)), ((), ())),
                           preferred_element_type=jnp.float32)
    es = jnp.sum(jnp.exp2(sim2), axis=1, keepdims=True)

    @pl.when(j == 0)
    def _():
        acc_ref[...] = es

    @pl.when(j > 0)
    def _():
        acc_ref[...] += es

    @pl.when(j == nj - 1)
    def _():
        p = p_ref[0]
        pn = p * lax.rsqrt(jnp.maximum(jnp.sum(p * p, axis=1, keepdims=True),
                                       1e-24))
        pos = jnp.sum(an * pn, axis=1, keepdims=True) * 2.0
        lse = jnp.log(acc_ref[...] + jnp.exp(pos))
        part = jnp.mean(lse - pos)
        s = jnp.where(l == 0, sig_ref[0, 0], sig_ref[0, 1])
        contrib = part / (2.0 * s * s) + jnp.log(s)

        contrib2d = jnp.reshape(contrib, (1, 1))

        @pl.when(l == 0)
        def _():
            out_ref[...] = contrib2d

        @pl.when(l == 1)
        def _():
            out_ref[...] = out_ref[...] + contrib2d


@functools.lru_cache(maxsize=None)
def _make_ntxent(B, K, D, CB=2048):
    BK = B * K
    assert BK % CB == 0
    nj = BK // CB
    return pl.pallas_call(
        _ntxent_body,
        grid=(2, nj),
        in_specs=[
            pl.BlockSpec((1, 2), lambda l, j: (0, 0)),
            pl.BlockSpec((1, B, D), lambda l, j: (l, 0, 0)),
            pl.BlockSpec((1, B, D), lambda l, j: (l, 0, 0)),
            pl.BlockSpec((1, CB, D), lambda l, j: (l, j, 0)),
        ],
        out_specs=pl.BlockSpec((1, 1), lambda l, j: (0, 0)),
        out_shape=jax.ShapeDtypeStruct((1, 1), jnp.float32),
        scratch_shapes=[pltpu.VMEM((B, 1), jnp.float32)],
        compiler_params=pltpu.CompilerParams(
            dimension_semantics=("arbitrary", "arbitrary")),
    )


def kernel(embed, sigma, anchor_class, pos_class, neg_class,
           anchor_ingred, pos_ingred, neg_ingred):
    D = embed.shape[1]
    B = anchor_class.shape[0]
    K = neg_class.shape[0] // B

    idx = jnp.concatenate([
        anchor_class, anchor_ingred, pos_class, pos_ingred,
        neg_class, neg_ingred,
    ])
    rows = _make_sc_gather(idx.shape[0], D)(embed, idx)

    A = rows[:2 * B].reshape(2, B, D)
    P = rows[2 * B:4 * B].reshape(2, B, D)
    N = rows[4 * B:].reshape(2, B * K, D)

    out = _make_ntxent(B, K, D)(sigma.reshape(1, 2), A, P, N)
    return out.reshape(())


# speedup vs baseline: 2.6004x; 1.0118x over previous
import functools

import jax
import jax.numpy as jnp
from jax import lax
from jax.experimental import pallas as pl
from jax.experimental.pallas import tpu as pltpu
from jax.experimental.pallas import tpu_sc as plsc


@functools.lru_cache(maxsize=None)
def _make_sc_gather(total, D):
    info = plsc.get_sparse_core_info()
    NC, NS = info.num_cores, info.num_subcores
    NW = NC * NS
    CH = 128
    assert total % (NW * CH) == 0
    n_g = total // (NW * CH)
    b_per_w = n_g * CH

    mesh = plsc.VectorSubcoreMesh(core_axis_name="c", subcore_axis_name="s")

    @functools.partial(
        pl.kernel,
        mesh=mesh,
        out_type=jax.ShapeDtypeStruct((total, D), jnp.float32),
        scratch_types=[
            pltpu.VMEM((n_g, CH), jnp.int32),
            pltpu.VMEM((CH, D), jnp.float32),
            pltpu.VMEM((CH, D), jnp.float32),
            pltpu.SemaphoreType.DMA,
            pltpu.SemaphoreType.DMA,
        ],
    )
    def gather_k(table_hbm, idx_hbm, out_hbm, idx_v, rows0, rows1, sem0, sem1):
        wid = lax.axis_index("s") * NC + lax.axis_index("c")
        base = wid * b_per_w
        pltpu.sync_copy(idx_hbm.at[wid], idx_v)
        bufs = (rows0, rows1)
        sems = (sem0, sem1)
        cps = [None, None]
        cps[0] = pltpu.async_copy(table_hbm.at[idx_v.at[0]], bufs[0], sems[0])
        for g in range(n_g):
            if g + 1 < n_g:
                cps[(g + 1) % 2] = pltpu.async_copy(
                    table_hbm.at[idx_v.at[g + 1]], bufs[(g + 1) % 2],
                    sems[(g + 1) % 2])
            cps[g % 2].wait()
            pltpu.sync_copy(bufs[g % 2], out_hbm.at[pl.ds(base + g * CH, CH)])

    def run(table, idx):
        idx3 = idx.reshape(NW, n_g, CH)
        return gather_k(table, idx3)

    return run


def _ntxent_body(sig_ref, a_ref, p_ref, n_ref, out_ref, acc_ref):
    l = pl.program_id(0)
    j = pl.program_id(1)
    nj = pl.num_programs(1)

    LOG2E2 = 2.0 * 1.4426950408889634
    a = a_ref[0]
    an = a * lax.rsqrt(jnp.maximum(jnp.sum(a * a, axis=1, keepdims=True),
                                   1e-24))
    nb = n_ref[0]
    nn = nb * lax.rsqrt(jnp.maximum(jnp.sum(nb * nb, axis=1, keepdims=True),
                                    1e-24))
    sim2 = lax.dot_general((an * LOG2E2).astype(jnp.bfloat16),
                           nn.astype(jnp.bfloat16), (((1,), (1,)), ((), ())),
                           preferred_element_type=jnp.float32)
    es = jnp.sum(jnp.exp2(sim2.astype(jnp.bfloat16)).astype(jnp.float32),
                 axis=1, keepdims=True)

    @pl.when(j == 0)
    def _():
        acc_ref[...] = es

    @pl.when(j > 0)
    def _():
        acc_ref[...] += es

    @pl.when(j == nj - 1)
    def _():
        p = p_ref[0]
        pn = p * lax.rsqrt(jnp.maximum(jnp.sum(p * p, axis=1, keepdims=True),
                                       1e-24))
        pos = jnp.sum(an * pn, axis=1, keepdims=True) * 2.0
        lse = jnp.log(acc_ref[...] + jnp.exp(pos))
        part = jnp.mean(lse - pos)
        s = jnp.where(l == 0, sig_ref[0, 0], sig_ref[0, 1])
        contrib = part / (2.0 * s * s) + jnp.log(s)

        contrib2d = jnp.reshape(contrib, (1, 1))

        @pl.when(l == 0)
        def _():
            out_ref[...] = contrib2d

        @pl.when(l == 1)
        def _():
            out_ref[...] = out_ref[...] + contrib2d


@functools.lru_cache(maxsize=None)
def _make_ntxent(B, K, D, CB=2048):
    BK = B * K
    assert BK % CB == 0
    nj = BK // CB
    return pl.pallas_call(
        _ntxent_body,
        grid=(2, nj),
        in_specs=[
            pl.BlockSpec((1, 2), lambda l, j: (0, 0)),
            pl.BlockSpec((1, B, D), lambda l, j: (l, 0, 0)),
            pl.BlockSpec((1, B, D), lambda l, j: (l, 0, 0)),
            pl.BlockSpec((1, CB, D), lambda l, j: (l, j, 0)),
        ],
        out_specs=pl.BlockSpec((1, 1), lambda l, j: (0, 0)),
        out_shape=jax.ShapeDtypeStruct((1, 1), jnp.float32),
        scratch_shapes=[pltpu.VMEM((B, 1), jnp.float32)],
        compiler_params=pltpu.CompilerParams(
            dimension_semantics=("arbitrary", "arbitrary")),
    )


def kernel(embed, sigma, anchor_class, pos_class, neg_class,
           anchor_ingred, pos_ingred, neg_ingred):
    D = embed.shape[1]
    B = anchor_class.shape[0]
    K = neg_class.shape[0] // B

    idx = jnp.concatenate([
        anchor_class, anchor_ingred, pos_class, pos_ingred,
        neg_class, neg_ingred,
    ])
    rows = _make_sc_gather(idx.shape[0], D)(embed, idx)

    A = rows[:2 * B].reshape(2, B, D)
    P = rows[2 * B:4 * B].reshape(2, B, D)
    N = rows[4 * B:].reshape(2, B * K, D)

    out = _make_ntxent(B, K, D)(sigma.reshape(1, 2), A, P, N)
    return out.reshape(())


# speedup vs baseline: 3.4780x; 1.3375x over previous
import functools

import jax
import jax.numpy as jnp
from jax import lax
from jax.experimental import pallas as pl
from jax.experimental.pallas import tpu as pltpu
from jax.experimental.pallas import tpu_sc as plsc

_LOG2E2 = 2.0 * 1.4426950408889634


@functools.lru_cache(maxsize=None)
def _make_sc_gather(total, D):
    info = plsc.get_sparse_core_info()
    NC, NS = info.num_cores, info.num_subcores
    NW = NC * NS
    CH = 128
    assert total % (NW * CH) == 0
    n_g = total // (NW * CH)
    b_per_w = n_g * CH

    mesh = plsc.VectorSubcoreMesh(core_axis_name="c", subcore_axis_name="s")

    @functools.partial(
        pl.kernel,
        mesh=mesh,
        out_type=jax.ShapeDtypeStruct((total, D), jnp.float32),
        scratch_types=[
            pltpu.VMEM((n_g, CH), jnp.int32),
            pltpu.VMEM((CH, D), jnp.float32),
            pltpu.VMEM((CH, D), jnp.float32),
            pltpu.SemaphoreType.DMA,
            pltpu.SemaphoreType.DMA,
        ],
    )
    def gather_k(table_hbm, idx_hbm, out_hbm, idx_v, rows0, rows1, sem0, sem1):
        wid = lax.axis_index("s") * NC + lax.axis_index("c")
        base = wid * b_per_w
        pltpu.sync_copy(idx_hbm.at[wid], idx_v)
        bufs = (rows0, rows1)
        sems = (sem0, sem1)
        cps = [None, None]
        cps[0] = pltpu.async_copy(table_hbm.at[idx_v.at[0]], bufs[0], sems[0])
        for g in range(n_g):
            if g + 1 < n_g:
                cps[(g + 1) % 2] = pltpu.async_copy(
                    table_hbm.at[idx_v.at[g + 1]], bufs[(g + 1) % 2],
                    sems[(g + 1) % 2])
            cps[g % 2].wait()
            pltpu.sync_copy(bufs[g % 2], out_hbm.at[pl.ds(base + g * CH, CH)])

    def run(table, idx):
        idx3 = idx.reshape(NW, n_g, CH)
        return gather_k(table, idx3)

    return run


def _loss_body(which, sig_ref, prev_ref, a_ref, p_ref, n_ref, out_ref,
               acc_ref, an_ref):
    j = pl.program_id(0)
    nj = pl.num_programs(0)

    @pl.when(j == 0)
    def _():
        a = a_ref[...]
        an = a * lax.rsqrt(jnp.maximum(jnp.sum(a * a, axis=1, keepdims=True),
                                       1e-24))
        an_ref[...] = (an * _LOG2E2).astype(jnp.bfloat16)

    nb = n_ref[...]
    nn = nb * lax.rsqrt(jnp.maximum(jnp.sum(nb * nb, axis=1, keepdims=True),
                                    1e-24))
    sim2 = lax.dot_general(an_ref[...], nn.astype(jnp.bfloat16),
                           (((1,), (1,)), ((), ())),
                           preferred_element_type=jnp.float32)
    es = jnp.sum(jnp.exp2(sim2.astype(jnp.bfloat16)).astype(jnp.float32),
                 axis=1, keepdims=True)

    @pl.when(j == 0)
    def _():
        acc_ref[...] = es

    @pl.when(j > 0)
    def _():
        acc_ref[...] += es

    @pl.when(j == nj - 1)
    def _():
        a = a_ref[...]
        an = a * lax.rsqrt(jnp.maximum(jnp.sum(a * a, axis=1, keepdims=True),
                                       1e-24))
        p = p_ref[...]
        pn = p * lax.rsqrt(jnp.maximum(jnp.sum(p * p, axis=1, keepdims=True),
                                       1e-24))
        pos = jnp.sum(an * pn, axis=1, keepdims=True) * 2.0
        lse = jnp.log(acc_ref[...] + jnp.exp(pos))
        part = jnp.mean(lse - pos)
        s = sig_ref[0, which]
        contrib = part / (2.0 * s * s) + jnp.log(s)
        out_ref[...] = prev_ref[...] + jnp.reshape(contrib, (1, 1))


@functools.lru_cache(maxsize=None)
def _make_loss(which, B, K, D, CB=2048):
    BK = B * K
    assert BK % CB == 0 and B == CB
    nj = BK // CB
    return pl.pallas_call(
        functools.partial(_loss_body, which),
        grid=(nj,),
        in_specs=[
            pl.BlockSpec((1, 2), lambda j: (0, 0)),
            pl.BlockSpec((1, 1), lambda j: (0, 0)),
            pl.BlockSpec((B, D), lambda j: (0, 0)),
            pl.BlockSpec((B, D), lambda j: (1, 0)),
            pl.BlockSpec((CB, D), lambda j: (2 + j, 0)),
        ],
        out_specs=pl.BlockSpec((1, 1), lambda j: (0, 0)),
        out_shape=jax.ShapeDtypeStruct((1, 1), jnp.float32),
        scratch_shapes=[
            pltpu.VMEM((B, 1), jnp.float32),
            pltpu.VMEM((B, D), jnp.bfloat16),
        ],
        compiler_params=pltpu.CompilerParams(
            dimension_semantics=("arbitrary",)),
    )


def kernel(embed, sigma, anchor_class, pos_class, neg_class,
           anchor_ingred, pos_ingred, neg_ingred):
    D = embed.shape[1]
    B = anchor_class.shape[0]
    K = neg_class.shape[0] // B

    idx_c = jnp.concatenate([anchor_class, pos_class, neg_class])
    idx_i = jnp.concatenate([anchor_ingred, pos_ingred, neg_ingred])
    gather = _make_sc_gather(idx_c.shape[0], D)
    rows_c = gather(embed, idx_c)
    rows_i = gather(embed, idx_i)

    sig2d = sigma.reshape(1, 2)
    zero = jnp.zeros((1, 1), jnp.float32)
    loss_c = _make_loss(0, B, K, D)(sig2d, zero, rows_c, rows_c, rows_c)
    total = _make_loss(1, B, K, D)(sig2d, loss_c, rows_i, rows_i, rows_i)
    return total.reshape(())


# speedup vs baseline: 3.6021x; 1.0357x over previous
import functools

import jax
import jax.numpy as jnp
from jax import lax
from jax.experimental import pallas as pl
from jax.experimental.pallas import tpu as pltpu
from jax.experimental.pallas import tpu_sc as plsc

_LOG2E2 = 2.0 * 1.4426950408889634


@functools.lru_cache(maxsize=None)
def _make_sc_gather(total, D):
    info = plsc.get_sparse_core_info()
    NC, NS = info.num_cores, info.num_subcores
    NW = NC * NS
    CH = 128
    assert total % (NW * CH) == 0
    n_g = total // (NW * CH)
    b_per_w = n_g * CH

    mesh = plsc.VectorSubcoreMesh(core_axis_name="c", subcore_axis_name="s")

    @functools.partial(
        pl.kernel,
        mesh=mesh,
        out_type=jax.ShapeDtypeStruct((total, D), jnp.float32),
        scratch_types=[
            pltpu.VMEM((n_g, CH), jnp.int32),
            pltpu.VMEM((CH, D), jnp.float32),
            pltpu.VMEM((CH, D), jnp.float32),
            pltpu.SemaphoreType.DMA,
            pltpu.SemaphoreType.DMA,
        ],
    )
    def gather_k(table_hbm, idx_hbm, out_hbm, idx_v, rows0, rows1, sem0, sem1):
        wid = lax.axis_index("s") * NC + lax.axis_index("c")
        base = wid * b_per_w
        pltpu.sync_copy(idx_hbm.at[wid], idx_v)
        bufs = (rows0, rows1)
        sems = (sem0, sem1)
        cps = [None, None]
        cps[0] = pltpu.async_copy(table_hbm.at[idx_v.at[0]], bufs[0], sems[0])
        for g in range(n_g):
            if g + 1 < n_g:
                cps[(g + 1) % 2] = pltpu.async_copy(
                    table_hbm.at[idx_v.at[g + 1]], bufs[(g + 1) % 2],
                    sems[(g + 1) % 2])
            cps[g % 2].wait()
            pltpu.sync_copy(bufs[g % 2], out_hbm.at[pl.ds(base + g * CH, CH)])

    def run(table, idx):
        idx3 = idx.reshape(NW, n_g, CH)
        return gather_k(table, idx3)

    return run


def _loss_body(which, sig_ref, prev_ref, a_ref, p_ref, n_ref, out_ref,
               acc_ref, an_ref):
    j = pl.program_id(0)
    nj = pl.num_programs(0)

    @pl.when(j == 0)
    def _():
        a = a_ref[...]
        an = a * lax.rsqrt(jnp.maximum(jnp.sum(a * a, axis=1, keepdims=True),
                                       1e-24))
        an_ref[...] = (an * _LOG2E2).astype(jnp.bfloat16)

    nb = n_ref[...]
    nn = nb * lax.rsqrt(jnp.maximum(jnp.sum(nb * nb, axis=1, keepdims=True),
                                    1e-24))
    sim2 = lax.dot_general(an_ref[...], nn.astype(jnp.bfloat16),
                           (((1,), (1,)), ((), ())),
                           preferred_element_type=jnp.float32)
    es = jnp.sum(jnp.exp2(sim2.astype(jnp.bfloat16)).astype(jnp.float32),
                 axis=1, keepdims=True)

    @pl.when(j == 0)
    def _():
        acc_ref[...] = es

    @pl.when(j > 0)
    def _():
        acc_ref[...] += es

    @pl.when(j == nj - 1)
    def _():
        a = a_ref[...]
        an = a * lax.rsqrt(jnp.maximum(jnp.sum(a * a, axis=1, keepdims=True),
                                       1e-24))
        p = p_ref[...]
        pn = p * lax.rsqrt(jnp.maximum(jnp.sum(p * p, axis=1, keepdims=True),
                                       1e-24))
        pos = jnp.sum(an * pn, axis=1, keepdims=True) * 2.0
        lse = jnp.log(acc_ref[...] + jnp.exp(pos))
        part = jnp.mean(lse - pos)
        s = sig_ref[0, which]
        contrib = part / (2.0 * s * s) + jnp.log(s)
        out_ref[...] = prev_ref[...] + jnp.reshape(contrib, (1, 1))


@functools.lru_cache(maxsize=None)
def _make_loss(which, B, K, D, CB=4096):
    BK = B * K
    assert BK % CB == 0 and (2 * B) % CB == 0
    nj = BK // CB
    noff = 2 * B // CB
    return pl.pallas_call(
        functools.partial(_loss_body, which),
        grid=(nj,),
        in_specs=[
            pl.BlockSpec((1, 2), lambda j: (0, 0)),
            pl.BlockSpec((1, 1), lambda j: (0, 0)),
            pl.BlockSpec((B, D), lambda j: (0, 0)),
            pl.BlockSpec((B, D), lambda j: (1, 0)),
            pl.BlockSpec((CB, D), lambda j: (noff + j, 0)),
        ],
        out_specs=pl.BlockSpec((1, 1), lambda j: (0, 0)),
        out_shape=jax.ShapeDtypeStruct((1, 1), jnp.float32),
        scratch_shapes=[
            pltpu.VMEM((B, 1), jnp.float32),
            pltpu.VMEM((B, D), jnp.bfloat16),
        ],
        compiler_params=pltpu.CompilerParams(
            dimension_semantics=("arbitrary",)),
    )


def kernel(embed, sigma, anchor_class, pos_class, neg_class,
           anchor_ingred, pos_ingred, neg_ingred):
    D = embed.shape[1]
    B = anchor_class.shape[0]
    K = neg_class.shape[0] // B

    idx_c = jnp.concatenate([anchor_class, pos_class, neg_class])
    idx_i = jnp.concatenate([anchor_ingred, pos_ingred, neg_ingred])
    gather = _make_sc_gather(idx_c.shape[0], D)
    rows_c = gather(embed, idx_c)
    rows_i = gather(embed, idx_i)

    sig2d = sigma.reshape(1, 2)
    zero = jnp.zeros((1, 1), jnp.float32)
    loss_c = _make_loss(0, B, K, D)(sig2d, zero, rows_c, rows_c, rows_c)
    total = _make_loss(1, B, K, D)(sig2d, loss_c, rows_i, rows_i, rows_i)
    return total.reshape(())
